# baseline (speedup 1.0000x reference)
import functools

import jax
import jax.numpy as jnp
from jax import lax
from jax.experimental import pallas as pl
from jax.experimental.pallas import tpu as pltpu
from jax.experimental.pallas import tpu_sc as plsc

NC, NS, L = 2, 16, 16
NW = NC * NS
HH = 128
K = 128
EP = 327680
KD = 100
NP = 10240
HALF = NP // 2
AR = 5248


def _make_deg(e):
    et = e // NS
    ch = et // KD
    nrt = NP // NS
    mesh = plsc.VectorSubcoreMesh(core_axis_name="c", subcore_axis_name="s")

    @functools.partial(
        pl.kernel,
        out_type=jax.ShapeDtypeStruct((NP,), jnp.float32),
        mesh=mesh,
        scratch_types=[
            pltpu.VMEM((ch, KD), jnp.int32),
            pltpu.VMEM((128,), jnp.float32),
            pltpu.VMEM((nrt,), jnp.float32),
            pltpu.VMEM_SHARED((NP,), jnp.float32),
        ],
    )
    def deg_kernel(dst_hbm, out_hbm, dst_v, ones_v, zero_v, acc_sh):
        cid = lax.axis_index("c")
        sid = lax.axis_index("s")
        pltpu.sync_copy(dst_hbm.at[sid], dst_v)
        zeros16 = jnp.zeros((L,), jnp.float32)
        ones16 = jnp.ones((L,), jnp.float32)

        def obody(i, carry):
            ones_v[pl.ds(pl.multiple_of(i * L, L), L)] = ones16
            return carry

        lax.fori_loop(0, 128 // L, obody, 0)

        def zbody(i, carry):
            zero_v[pl.ds(pl.multiple_of(i * L, L), L)] = zeros16
            return carry

        lax.fori_loop(0, nrt // L, zbody, 0)
        pltpu.sync_copy(zero_v, acc_sh.at[pl.ds(sid * nrt, nrt)])
        plsc.subcore_barrier()

        def body(j, carry):
            pltpu.sync_copy(ones_v.at[pl.ds(0, KD)], acc_sh.at[dst_v.at[j]],
                            add=True)
            return carry

        lax.fori_loop(0, ch, body, 0)
        plsc.subcore_barrier()

        @pl.when(cid == 0)
        def _():
            pltpu.sync_copy(acc_sh.at[pl.ds(sid * nrt, nrt)], zero_v)
            pltpu.sync_copy(zero_v, out_hbm.at[pl.ds(sid * nrt, nrt)])

    return deg_kernel


def _make_conv(e):
    et = e // NS
    ch = et // K
    art = AR // NS
    wrt = HALF // NS
    mesh = plsc.VectorSubcoreMesh(core_axis_name="c", subcore_axis_name="s")

    @functools.partial(
        pl.kernel,
        out_type=(
            jax.ShapeDtypeStruct((NP, HH), jnp.float32),
            jax.ShapeDtypeStruct((NP, HH), jnp.float32),
        ),
        mesh=mesh,
        scratch_types=[
            pltpu.VMEM((ch, K), jnp.int32),
            pltpu.VMEM((ch, K), jnp.int32),
            pltpu.VMEM((K, HH), jnp.float32),
            pltpu.VMEM((K, HH), jnp.float32),
            pltpu.VMEM_SHARED((AR, HH), jnp.float32),
            pltpu.SemaphoreType.DMA,
            pltpu.SemaphoreType.DMA,
            pltpu.SemaphoreType.DMA,
            pltpu.SemaphoreType.DMA,
        ],
    )
    def conv_kernel(lo_hbm, hi_hbm, src_hbm, adj0_hbm, adj1_hbm,
                    out_lo, out_hi,
                    src_v, adj_v, rows0_v, rows1_v, acc_sh,
                    sem0, sem1, sem2, sem3):
        cid = lax.axis_index("c")
        sid = lax.axis_index("s")
        pltpu.sync_copy(src_hbm.at[sid], src_v)

        zeros16 = jnp.zeros((L,), jnp.float32)

        def run(table_hbm, out_hbm):
            for p, adj_hbm in ((0, adj0_hbm), (1, adj1_hbm)):
                base = p * HALF
                pltpu.sync_copy(adj_hbm.at[sid], adj_v)

                def zbody(i, carry):
                    for j in range(HH // L):
                        rows0_v[i, pl.ds(j * L, L)] = zeros16
                    return carry

                lax.fori_loop(0, 8, zbody, 0)

                def zc_body(t, carry):
                    off = pl.multiple_of(t * 8, 8)
                    pltpu.sync_copy(rows0_v.at[pl.ds(0, 8)],
                                    acc_sh.at[pl.ds(sid * art + off, 8)])
                    return carry

                lax.fori_loop(0, art // 8, zc_body, 0)
                plsc.subcore_barrier()

                pltpu.async_copy(table_hbm.at[src_v.at[0]], rows0_v, sem0)

                def body(jj, carry):
                    j0 = jj * 2
                    pltpu.async_copy(table_hbm.at[src_v.at[j0 + 1]],
                                     rows1_v, sem1)
                    pltpu.make_async_copy(table_hbm.at[src_v.at[j0]],
                                          rows0_v, sem0).wait()
                    pltpu.async_copy(rows0_v, acc_sh.at[adj_v.at[j0]],
                                     sem2, add=True)
                    pltpu.make_async_copy(table_hbm.at[src_v.at[j0 + 1]],
                                          rows1_v, sem1).wait()
                    pltpu.async_copy(rows1_v, acc_sh.at[adj_v.at[j0 + 1]],
                                     sem3, add=True)
                    pltpu.make_async_copy(rows0_v, acc_sh.at[adj_v.at[j0]],
                                          sem2).wait()

                    @pl.when(j0 + 2 < ch)
                    def _():
                        pltpu.async_copy(table_hbm.at[src_v.at[j0 + 2]],
                                         rows0_v, sem0)

                    pltpu.make_async_copy(rows1_v,
                                          acc_sh.at[adj_v.at[j0 + 1]],
                                          sem3).wait()
                    return carry

                lax.fori_loop(0, ch // 2, body, 0)
                plsc.subcore_barrier()

                def wb_body(t, carry):
                    off = pl.multiple_of(t * 64, 8)
                    pltpu.sync_copy(acc_sh.at[pl.ds(sid * wrt + off, 64)],
                                    rows1_v.at[pl.ds(0, 64)])
                    pltpu.sync_copy(rows1_v.at[pl.ds(0, 64)],
                                    out_hbm.at[pl.ds(base + sid * wrt + off,
                                                     64)])
                    return carry

                lax.fori_loop(0, wrt // 64, wb_body, 0)
                plsc.subcore_barrier()

        @pl.when(cid == 0)
        def _():
            run(lo_hbm, out_lo)

        @pl.when(cid == 1)
        def _():
            run(hi_hbm, out_hi)

    return conv_kernel


def _tc1_body(nblocks, x_ref, degc_ref, dstb_ref, w1, b1, w2, b2, w3, b3,
              wc1, h_out, lo_out, hi_out, adj0_out, adj1_out,
              hsum_out, hmax_out, msum, mmax):
    i = pl.program_id(0)
    nrm = lax.rsqrt(degc_ref[...] + 1.0)
    dv = dstb_ref[...]
    oob0 = (dv < 0) | (dv >= HALF)
    adj0_out[...] = jnp.where(oob0, HALF, dv)
    d1 = dv - HALF
    oob1 = (d1 < 0) | (d1 >= HALF)
    adj1_out[...] = jnp.where(oob1, HALF, d1)
    h = jnp.tanh(jnp.dot(x_ref[...], w1[...],
                         preferred_element_type=jnp.float32) + b1[...])
    h = jnp.tanh(jnp.dot(h, w2[...],
                         preferred_element_type=jnp.float32) + b2[...])
    h = jnp.tanh(jnp.dot(h, w3[...],
                         preferred_element_type=jnp.float32) + b3[...])
    hw = jnp.dot(h, wc1[...], preferred_element_type=jnp.float32)
    hwn = hw * nrm
    h_out[...] = h
    lo_out[...] = hwn[:, :HH]
    hi_out[...] = hwn[:, HH:]
    psum = jnp.sum(h, axis=0, keepdims=True)
    pmax = jnp.max(h, axis=0, keepdims=True)

    @pl.when(i == 0)
    def _():
        msum[...] = psum
        mmax[...] = pmax

    @pl.when(i > 0)
    def _():
        msum[...] = msum[...] + psum
        mmax[...] = jnp.maximum(mmax[...], pmax)

    @pl.when(i == nblocks - 1)
    def _():
        hsum_out[...] = msum[...]
        hmax_out[...] = mmax[...]


def _tc2_body(nblocks, s_lo, s_hi, g_lo, g_hi, h_ref, degc_ref, wa, wb,
              bias, lo_out, hi_out, csum_out, cmax_out, msum, mmax):
    i = pl.program_id(0)
    nrm = lax.rsqrt(degc_ref[...] + 1.0)
    s = jnp.concatenate([s_lo[...], s_hi[...]], axis=1)
    g = jnp.concatenate([g_lo[...], g_hi[...]], axis=1)
    c = jnp.tanh(nrm * (s + g) + bias[...])
    hw = (jnp.dot(c, wa[...], preferred_element_type=jnp.float32)
          + jnp.dot(h_ref[...], wb[...],
                    preferred_element_type=jnp.float32))
    hwn = hw * nrm
    lo_out[...] = hwn[:, :HH]
    hi_out[...] = hwn[:, HH:]
    psum = jnp.sum(c, axis=0, keepdims=True)
    pmax = jnp.max(c, axis=0, keepdims=True)

    @pl.when(i == 0)
    def _():
        msum[...] = psum
        mmax[...] = pmax

    @pl.when(i > 0)
    def _():
        msum[...] = msum[...] + psum
        mmax[...] = jnp.maximum(mmax[...], pmax)

    @pl.when(i == nblocks - 1)
    def _():
        csum_out[...] = msum[...]
        cmax_out[...] = mmax[...]


def _tc3_body(n_nodes, c1sum, c1max, c2sum, c2max, hsum_ref, hmax_ref,
              wp1, bp1, wp2, bp2, z_out):
    inv_n = 1.0 / n_nodes
    ro = jnp.concatenate(
        [c2sum[...] * inv_n, c1sum[...] * inv_n, hsum_ref[...] * inv_n,
         c2max[...], c1max[...], hmax_ref[...]], axis=1)
    z1 = jnp.tanh(jnp.dot(ro, wp1[...],
                          preferred_element_type=jnp.float32) + bp1[...])
    z_out[...] = jnp.dot(z1, wp2[...],
                         preferred_element_type=jnp.float32) + bp2[...]


def kernel(x, edge_index, W_pre1, b_pre1, W_pre2, b_pre2, W_pre3, b_pre3,
           W_c1, b_c1, W_c2, b_c2, W_p1, b_p1, W_p2, b_p2):
    n, d = x.shape
    e = edge_index.shape[1]
    hid = W_pre1.shape[1]
    out_dim = W_p2.shape[1]
    R = 1000
    nb = n // R

    src = edge_index[0]
    dst = edge_index[1]
    pad = EP - e
    src_p = jnp.concatenate([src, jnp.zeros((pad,), jnp.int32)])
    dst_p = jnp.concatenate([dst, jnp.full((pad,), -1, jnp.int32)])
    src_t = src_p.reshape(NS, (EP // NS) // K, K)
    dst_b = dst_p.reshape(EP // 128, 128)
    dst_d = dst.reshape(NS, (e // NS) // KD, KD)
    eb = (EP // 128) // nb

    degc = _make_deg(e)(dst_d).reshape(NP, 1)[:n]

    b1 = b_pre1.reshape(1, hid)
    b2 = b_pre2.reshape(1, hid)
    b3 = b_pre3.reshape(1, hid)
    bc1 = b_c1.reshape(1, hid)
    bc2 = b_c2.reshape(1, hid)
    bp1 = b_p1.reshape(1, hid)
    bp2 = b_p2.reshape(1, out_dim)
    wc2a = W_c2[:hid]
    wc2b = W_c2[hid:]

    full = lambda a: pl.BlockSpec(a.shape, lambda i: (0,) * a.ndim)
    rows = lambda c: pl.BlockSpec((R, c), lambda i: (i, 0))
    degs = pl.BlockSpec((R, 1), lambda i: (i, 0))
    stat = pl.BlockSpec((1, hid), lambda i: (0, 0))

    edg = pl.BlockSpec((eb, 128), lambda i: (i, 0))
    res = pl.pallas_call(
        functools.partial(_tc1_body, nb),
        grid=(nb,),
        in_specs=[rows(d), degs, edg, full(W_pre1), full(b1), full(W_pre2),
                  full(b2), full(W_pre3), full(b3), full(W_c1)],
        out_specs=[rows(hid), rows(HH), rows(HH), edg, edg, stat, stat],
        out_shape=[jax.ShapeDtypeStruct((n, hid), jnp.float32),
                   jax.ShapeDtypeStruct((n, HH), jnp.float32),
                   jax.ShapeDtypeStruct((n, HH), jnp.float32),
                   jax.ShapeDtypeStruct((EP // 128, 128), jnp.int32),
                   jax.ShapeDtypeStruct((EP // 128, 128), jnp.int32),
                   jax.ShapeDtypeStruct((1, hid), jnp.float32),
                   jax.ShapeDtypeStruct((1, hid), jnp.float32)],
        scratch_shapes=[pltpu.VMEM((1, hid), jnp.float32),
                        pltpu.VMEM((1, hid), jnp.float32)],
    )(x, degc, dst_b, W_pre1, b1, W_pre2, b2, W_pre3, b3, W_c1)
    h, g1_lo, g1_hi, adj0, adj1, hsum, hmax = res
    adj0 = adj0.reshape(NS, (EP // NS) // K, K)
    adj1 = adj1.reshape(NS, (EP // NS) // K, K)

    conv = _make_conv(EP)

    tc2 = pl.pallas_call(
        functools.partial(_tc2_body, nb),
        grid=(nb,),
        in_specs=[rows(HH), rows(HH), rows(HH), rows(HH), rows(hid), degs,
                  full(wc2a), full(wc2b), stat],
        out_specs=[rows(HH), rows(HH), stat, stat],
        out_shape=[jax.ShapeDtypeStruct((n, HH), jnp.float32),
                   jax.ShapeDtypeStruct((n, HH), jnp.float32),
                   jax.ShapeDtypeStruct((1, hid), jnp.float32),
                   jax.ShapeDtypeStruct((1, hid), jnp.float32)],
        scratch_shapes=[pltpu.VMEM((1, hid), jnp.float32),
                        pltpu.VMEM((1, hid), jnp.float32)],
    )

    s1_lo, s1_hi = conv(g1_lo, g1_hi, src_t, adj0, adj1)
    g2_lo, g2_hi, c1sum, c1max = tc2(s1_lo, s1_hi, g1_lo, g1_hi, h, degc,
                                     wc2a, wc2b, bc1)
    s2_lo, s2_hi = conv(g2_lo, g2_hi, src_t, adj0, adj1)
    _, _, c2sum, c2max = tc2(s2_lo, s2_hi, g2_lo, g2_hi, h, degc,
                             wc2a, wc2b, bc2)

    z = pl.pallas_call(
        functools.partial(_tc3_body, float(n)),
        grid=(1,),
        in_specs=[stat, stat, stat, stat, stat, stat, full(W_p1), full(bp1),
                  full(W_p2), full(bp2)],
        out_specs=[pl.BlockSpec((1, out_dim), lambda i: (0, 0))],
        out_shape=[jax.ShapeDtypeStruct((1, out_dim), jnp.float32)],
    )(c1sum, c1max, c2sum, c2max, hsum, hmax, W_p1, bp1, W_p2, bp2)[0]

    return z[0].reshape(out_dim)


# speedup vs baseline: 1.1180x; 1.1180x over previous
import functools

import jax
import jax.numpy as jnp
from jax import lax
from jax.experimental import pallas as pl
from jax.experimental.pallas import tpu as pltpu
from jax.experimental.pallas import tpu_sc as plsc

NC, NS, L = 2, 16, 16
NW = NC * NS
HH = 128
K = 128
EP = 327680
KD = 100
NP = 10240
HALF = NP // 2
AR = 5248


def _make_deg(e):
    et = e // NS
    ch = et // KD
    nrt = NP // NS
    mesh = plsc.VectorSubcoreMesh(core_axis_name="c", subcore_axis_name="s")

    @functools.partial(
        pl.kernel,
        out_type=jax.ShapeDtypeStruct((NP,), jnp.float32),
        mesh=mesh,
        scratch_types=[
            pltpu.VMEM((ch, KD), jnp.int32),
            pltpu.VMEM((128,), jnp.float32),
            pltpu.VMEM((nrt,), jnp.float32),
            pltpu.VMEM_SHARED((NP,), jnp.float32),
        ],
    )
    def deg_kernel(dst_hbm, out_hbm, dst_v, ones_v, zero_v, acc_sh):
        cid = lax.axis_index("c")
        sid = lax.axis_index("s")
        pltpu.sync_copy(dst_hbm.at[sid], dst_v)
        zeros16 = jnp.zeros((L,), jnp.float32)
        ones16 = jnp.ones((L,), jnp.float32)

        def obody(i, carry):
            ones_v[pl.ds(pl.multiple_of(i * L, L), L)] = ones16
            return carry

        lax.fori_loop(0, 128 // L, obody, 0)

        def zbody(i, carry):
            zero_v[pl.ds(pl.multiple_of(i * L, L), L)] = zeros16
            return carry

        lax.fori_loop(0, nrt // L, zbody, 0)
        pltpu.sync_copy(zero_v, acc_sh.at[pl.ds(sid * nrt, nrt)])
        plsc.subcore_barrier()

        def body(j, carry):
            pltpu.sync_copy(ones_v.at[pl.ds(0, KD)], acc_sh.at[dst_v.at[j]],
                            add=True)
            return carry

        lax.fori_loop(0, ch, body, 0)
        plsc.subcore_barrier()

        @pl.when(cid == 0)
        def _():
            pltpu.sync_copy(acc_sh.at[pl.ds(sid * nrt, nrt)], zero_v)
            pltpu.sync_copy(zero_v, out_hbm.at[pl.ds(sid * nrt, nrt)])

    return deg_kernel


def _make_conv(e):
    et = e // NS
    ch = et // K
    art = AR // NS
    wrt = HALF // NS
    mesh = plsc.VectorSubcoreMesh(core_axis_name="c", subcore_axis_name="s")

    @functools.partial(
        pl.kernel,
        out_type=(
            jax.ShapeDtypeStruct((NP, HH), jnp.float32),
            jax.ShapeDtypeStruct((NP, HH), jnp.float32),
        ),
        mesh=mesh,
        scratch_types=[
            pltpu.VMEM((ch, K), jnp.int32),
            pltpu.VMEM((ch, K), jnp.int32),
            pltpu.VMEM((K, HH), jnp.float32),
            pltpu.VMEM((K, HH), jnp.float32),
            pltpu.VMEM_SHARED((AR, HH), jnp.float32),
            pltpu.SemaphoreType.DMA,
            pltpu.SemaphoreType.DMA,
            pltpu.SemaphoreType.DMA,
            pltpu.SemaphoreType.DMA,
        ],
    )
    def conv_kernel(lo_hbm, hi_hbm, src_hbm, adj0_hbm, adj1_hbm,
                    out_lo, out_hi,
                    src_v, adj_v, rows0_v, rows1_v, acc_sh,
                    sem0, sem1, sem2, sem3):
        cid = lax.axis_index("c")
        sid = lax.axis_index("s")
        pltpu.sync_copy(src_hbm.at[sid], src_v)

        zeros16 = jnp.zeros((L,), jnp.float32)

        def run(table_hbm, out_hbm):
            for p, adj_hbm in ((0, adj0_hbm), (1, adj1_hbm)):
                base = p * HALF
                pltpu.sync_copy(adj_hbm.at[sid], adj_v)

                def zbody(i, carry):
                    for j in range(HH // L):
                        rows0_v[i, pl.ds(j * L, L)] = zeros16
                    return carry

                lax.fori_loop(0, 8, zbody, 0)

                def zc_body(t, carry):
                    off = pl.multiple_of(t * 8, 8)
                    pltpu.sync_copy(rows0_v.at[pl.ds(0, 8)],
                                    acc_sh.at[pl.ds(sid * art + off, 8)])
                    return carry

                lax.fori_loop(0, art // 8, zc_body, 0)
                plsc.subcore_barrier()

                pltpu.async_copy(table_hbm.at[src_v.at[0]], rows0_v, sem0)

                def body(jj, carry):
                    j0 = jj * 2
                    pltpu.async_copy(table_hbm.at[src_v.at[j0 + 1]],
                                     rows1_v, sem1)
                    pltpu.make_async_copy(table_hbm.at[src_v.at[j0]],
                                          rows0_v, sem0).wait()
                    pltpu.sync_copy(rows0_v, acc_sh.at[adj_v.at[j0]],
                                    add=True)

                    @pl.when(j0 + 2 < ch)
                    def _():
                        pltpu.async_copy(table_hbm.at[src_v.at[j0 + 2]],
                                         rows0_v, sem0)

                    pltpu.make_async_copy(table_hbm.at[src_v.at[j0 + 1]],
                                          rows1_v, sem1).wait()
                    pltpu.sync_copy(rows1_v, acc_sh.at[adj_v.at[j0 + 1]],
                                    add=True)
                    return carry

                lax.fori_loop(0, ch // 2, body, 0)
                plsc.subcore_barrier()

                def wb_body(t, carry):
                    off = pl.multiple_of(t * 64, 8)
                    pltpu.sync_copy(acc_sh.at[pl.ds(sid * wrt + off, 64)],
                                    rows1_v.at[pl.ds(0, 64)])
                    pltpu.sync_copy(rows1_v.at[pl.ds(0, 64)],
                                    out_hbm.at[pl.ds(base + sid * wrt + off,
                                                     64)])
                    return carry

                lax.fori_loop(0, wrt // 64, wb_body, 0)
                plsc.subcore_barrier()

        @pl.when(cid == 0)
        def _():
            run(lo_hbm, out_lo)

        @pl.when(cid == 1)
        def _():
            run(hi_hbm, out_hi)

    return conv_kernel


def _tc1_body(nblocks, x_ref, degc_ref, dstb_ref, w1, b1, w2, b2, w3, b3,
              wc1, h_out, lo_out, hi_out, adj0_out, adj1_out,
              hsum_out, hmax_out, msum, mmax):
    i = pl.program_id(0)
    nrm = lax.rsqrt(degc_ref[...] + 1.0)
    dv = dstb_ref[...]
    trash = HALF + lax.broadcasted_iota(jnp.int32, dv.shape, 1)
    oob0 = (dv < 0) | (dv >= HALF)
    adj0_out[...] = jnp.where(oob0, trash, dv)
    d1 = dv - HALF
    oob1 = (d1 < 0) | (d1 >= HALF)
    adj1_out[...] = jnp.where(oob1, trash, d1)
    h = jnp.tanh(jnp.dot(x_ref[...], w1[...],
                         preferred_element_type=jnp.float32) + b1[...])
    h = jnp.tanh(jnp.dot(h, w2[...],
                         preferred_element_type=jnp.float32) + b2[...])
    h = jnp.tanh(jnp.dot(h, w3[...],
                         preferred_element_type=jnp.float32) + b3[...])
    hw = jnp.dot(h, wc1[...], preferred_element_type=jnp.float32)
    hwn = hw * nrm
    h_out[...] = h
    lo_out[...] = hwn[:, :HH]
    hi_out[...] = hwn[:, HH:]
    psum = jnp.sum(h, axis=0, keepdims=True)
    pmax = jnp.max(h, axis=0, keepdims=True)

    @pl.when(i == 0)
    def _():
        msum[...] = psum
        mmax[...] = pmax

    @pl.when(i > 0)
    def _():
        msum[...] = msum[...] + psum
        mmax[...] = jnp.maximum(mmax[...], pmax)

    @pl.when(i == nblocks - 1)
    def _():
        hsum_out[...] = msum[...]
        hmax_out[...] = mmax[...]


def _tc2_body(nblocks, s_lo, s_hi, g_lo, g_hi, h_ref, degc_ref, wa, wb,
              bias, lo_out, hi_out, csum_out, cmax_out, msum, mmax):
    i = pl.program_id(0)
    nrm = lax.rsqrt(degc_ref[...] + 1.0)
    s = jnp.concatenate([s_lo[...], s_hi[...]], axis=1)
    g = jnp.concatenate([g_lo[...], g_hi[...]], axis=1)
    c = jnp.tanh(nrm * (s + g) + bias[...])
    hw = (jnp.dot(c, wa[...], preferred_element_type=jnp.float32)
          + jnp.dot(h_ref[...], wb[...],
                    preferred_element_type=jnp.float32))
    hwn = hw * nrm
    lo_out[...] = hwn[:, :HH]
    hi_out[...] = hwn[:, HH:]
    psum = jnp.sum(c, axis=0, keepdims=True)
    pmax = jnp.max(c, axis=0, keepdims=True)

    @pl.when(i == 0)
    def _():
        msum[...] = psum
        mmax[...] = pmax

    @pl.when(i > 0)
    def _():
        msum[...] = msum[...] + psum
        mmax[...] = jnp.maximum(mmax[...], pmax)

    @pl.when(i == nblocks - 1)
    def _():
        csum_out[...] = msum[...]
        cmax_out[...] = mmax[...]


def _tc3_body(n_nodes, c1sum, c1max, c2sum, c2max, hsum_ref, hmax_ref,
              wp1, bp1, wp2, bp2, z_out):
    inv_n = 1.0 / n_nodes
    ro = jnp.concatenate(
        [c2sum[...] * inv_n, c1sum[...] * inv_n, hsum_ref[...] * inv_n,
         c2max[...], c1max[...], hmax_ref[...]], axis=1)
    z1 = jnp.tanh(jnp.dot(ro, wp1[...],
                          preferred_element_type=jnp.float32) + bp1[...])
    z_out[...] = jnp.dot(z1, wp2[...],
                         preferred_element_type=jnp.float32) + bp2[...]


def kernel(x, edge_index, W_pre1, b_pre1, W_pre2, b_pre2, W_pre3, b_pre3,
           W_c1, b_c1, W_c2, b_c2, W_p1, b_p1, W_p2, b_p2):
    n, d = x.shape
    e = edge_index.shape[1]
    hid = W_pre1.shape[1]
    out_dim = W_p2.shape[1]
    R = 1000
    nb = n // R

    src = edge_index[0]
    dst = edge_index[1]
    pad = EP - e
    src_p = jnp.concatenate([src, jnp.zeros((pad,), jnp.int32)])
    dst_p = jnp.concatenate([dst, jnp.full((pad,), -1, jnp.int32)])
    src_t = src_p.reshape(NS, (EP // NS) // K, K)
    dst_b = dst_p.reshape(EP // 128, 128)
    dst_d = dst.reshape(NS, (e // NS) // KD, KD)
    eb = (EP // 128) // nb

    degc = _make_deg(e)(dst_d).reshape(NP, 1)[:n]

    b1 = b_pre1.reshape(1, hid)
    b2 = b_pre2.reshape(1, hid)
    b3 = b_pre3.reshape(1, hid)
    bc1 = b_c1.reshape(1, hid)
    bc2 = b_c2.reshape(1, hid)
    bp1 = b_p1.reshape(1, hid)
    bp2 = b_p2.reshape(1, out_dim)
    wc2a = W_c2[:hid]
    wc2b = W_c2[hid:]

    full = lambda a: pl.BlockSpec(a.shape, lambda i: (0,) * a.ndim)
    rows = lambda c: pl.BlockSpec((R, c), lambda i: (i, 0))
    degs = pl.BlockSpec((R, 1), lambda i: (i, 0))
    stat = pl.BlockSpec((1, hid), lambda i: (0, 0))

    edg = pl.BlockSpec((eb, 128), lambda i: (i, 0))
    res = pl.pallas_call(
        functools.partial(_tc1_body, nb),
        grid=(nb,),
        in_specs=[rows(d), degs, edg, full(W_pre1), full(b1), full(W_pre2),
                  full(b2), full(W_pre3), full(b3), full(W_c1)],
        out_specs=[rows(hid), rows(HH), rows(HH), edg, edg, stat, stat],
        out_shape=[jax.ShapeDtypeStruct((n, hid), jnp.float32),
                   jax.ShapeDtypeStruct((n, HH), jnp.float32),
                   jax.ShapeDtypeStruct((n, HH), jnp.float32),
                   jax.ShapeDtypeStruct((EP // 128, 128), jnp.int32),
                   jax.ShapeDtypeStruct((EP // 128, 128), jnp.int32),
                   jax.ShapeDtypeStruct((1, hid), jnp.float32),
                   jax.ShapeDtypeStruct((1, hid), jnp.float32)],
        scratch_shapes=[pltpu.VMEM((1, hid), jnp.float32),
                        pltpu.VMEM((1, hid), jnp.float32)],
    )(x, degc, dst_b, W_pre1, b1, W_pre2, b2, W_pre3, b3, W_c1)
    h, g1_lo, g1_hi, adj0, adj1, hsum, hmax = res
    adj0 = adj0.reshape(NS, (EP // NS) // K, K)
    adj1 = adj1.reshape(NS, (EP // NS) // K, K)

    conv = _make_conv(EP)

    tc2 = pl.pallas_call(
        functools.partial(_tc2_body, nb),
        grid=(nb,),
        in_specs=[rows(HH), rows(HH), rows(HH), rows(HH), rows(hid), degs,
                  full(wc2a), full(wc2b), stat],
        out_specs=[rows(HH), rows(HH), stat, stat],
        out_shape=[jax.ShapeDtypeStruct((n, HH), jnp.float32),
                   jax.ShapeDtypeStruct((n, HH), jnp.float32),
                   jax.ShapeDtypeStruct((1, hid), jnp.float32),
                   jax.ShapeDtypeStruct((1, hid), jnp.float32)],
        scratch_shapes=[pltpu.VMEM((1, hid), jnp.float32),
                        pltpu.VMEM((1, hid), jnp.float32)],
    )

    s1_lo, s1_hi = conv(g1_lo, g1_hi, src_t, adj0, adj1)
    g2_lo, g2_hi, c1sum, c1max = tc2(s1_lo, s1_hi, g1_lo, g1_hi, h, degc,
                                     wc2a, wc2b, bc1)
    s2_lo, s2_hi = conv(g2_lo, g2_hi, src_t, adj0, adj1)
    _, _, c2sum, c2max = tc2(s2_lo, s2_hi, g2_lo, g2_hi, h, degc,
                             wc2a, wc2b, bc2)

    z = pl.pallas_call(
        functools.partial(_tc3_body, float(n)),
        grid=(1,),
        in_specs=[stat, stat, stat, stat, stat, stat, full(W_p1), full(bp1),
                  full(W_p2), full(bp2)],
        out_specs=[pl.BlockSpec((1, out_dim), lambda i: (0, 0))],
        out_shape=[jax.ShapeDtypeStruct((1, out_dim), jnp.float32)],
    )(c1sum, c1max, c2sum, c2max, hsum, hmax, W_p1, bp1, W_p2, bp2)[0]

    return z[0].reshape(out_dim)


# speedup vs baseline: 1.1211x; 1.0027x over previous
import functools

import jax
import jax.numpy as jnp
from jax import lax
from jax.experimental import pallas as pl
from jax.experimental.pallas import tpu as pltpu
from jax.experimental.pallas import tpu_sc as plsc

NC, NS, L = 2, 16, 16
NW = NC * NS
HH = 128
K = 128
EP = 327680
KD = 100
NP = 10240
HALF = NP // 2
AR = 5248


def _make_deg(e):
    et = e // NS
    ch = et // KD
    nrt = NP // NS
    mesh = plsc.VectorSubcoreMesh(core_axis_name="c", subcore_axis_name="s")

    @functools.partial(
        pl.kernel,
        out_type=jax.ShapeDtypeStruct((NP,), jnp.float32),
        mesh=mesh,
        scratch_types=[
            pltpu.VMEM((ch, KD), jnp.int32),
            pltpu.VMEM((128,), jnp.float32),
            pltpu.VMEM((nrt,), jnp.float32),
            pltpu.VMEM_SHARED((NP,), jnp.float32),
        ],
    )
    def deg_kernel(dst_hbm, out_hbm, dst_v, ones_v, zero_v, acc_sh):
        cid = lax.axis_index("c")
        sid = lax.axis_index("s")
        pltpu.sync_copy(dst_hbm.at[sid], dst_v)
        zeros16 = jnp.zeros((L,), jnp.float32)
        ones16 = jnp.ones((L,), jnp.float32)

        def obody(i, carry):
            ones_v[pl.ds(pl.multiple_of(i * L, L), L)] = ones16
            return carry

        lax.fori_loop(0, 128 // L, obody, 0)

        def zbody(i, carry):
            zero_v[pl.ds(pl.multiple_of(i * L, L), L)] = zeros16
            return carry

        lax.fori_loop(0, nrt // L, zbody, 0)
        pltpu.sync_copy(zero_v, acc_sh.at[pl.ds(sid * nrt, nrt)])
        plsc.subcore_barrier()

        def body(j, carry):
            pltpu.sync_copy(ones_v.at[pl.ds(0, KD)], acc_sh.at[dst_v.at[j]],
                            add=True)
            return carry

        lax.fori_loop(0, ch, body, 0)
        plsc.subcore_barrier()

        @pl.when(cid == 0)
        def _():
            pltpu.sync_copy(acc_sh.at[pl.ds(sid * nrt, nrt)], zero_v)
            pltpu.sync_copy(zero_v, out_hbm.at[pl.ds(sid * nrt, nrt)])

    return deg_kernel


def _make_conv(e):
    et = e // NS
    ch = et // K
    art = AR // NS
    wrt = HALF // NS
    mesh = plsc.VectorSubcoreMesh(core_axis_name="c", subcore_axis_name="s")

    @functools.partial(
        pl.kernel,
        out_type=(
            jax.ShapeDtypeStruct((NP, HH), jnp.float32),
            jax.ShapeDtypeStruct((NP, HH), jnp.float32),
        ),
        mesh=mesh,
        scratch_types=[
            pltpu.VMEM((ch, K), jnp.int32),
            pltpu.VMEM((ch, K), jnp.int32),
            pltpu.VMEM((K, HH), jnp.float32),
            pltpu.VMEM((K, HH), jnp.float32),
            pltpu.VMEM_SHARED((AR, HH), jnp.float32),
            pltpu.SemaphoreType.DMA,
            pltpu.SemaphoreType.DMA,
            pltpu.SemaphoreType.DMA,
            pltpu.SemaphoreType.DMA,
        ],
    )
    def conv_kernel(lo_hbm, hi_hbm, src_hbm, adj0_hbm, adj1_hbm,
                    out_lo, out_hi,
                    src_v, adj_v, rows0_v, rows1_v, acc_sh,
                    sem0, sem1, sem2, sem3):
        cid = lax.axis_index("c")
        sid = lax.axis_index("s")
        pltpu.sync_copy(src_hbm.at[sid], src_v)

        zeros16 = jnp.zeros((L,), jnp.float32)

        def run(table_hbm, out_hbm):
            for p, adj_hbm in ((0, adj0_hbm), (1, adj1_hbm)):
                base = p * HALF
                pltpu.sync_copy(adj_hbm.at[sid], adj_v)

                def zbody(i, carry):
                    for j in range(HH // L):
                        rows0_v[i, pl.ds(j * L, L)] = zeros16
                    return carry

                lax.fori_loop(0, 64, zbody, 0)

                def zc_body(t, carry):
                    off = pl.multiple_of(t * 64, 8)
                    pltpu.sync_copy(rows0_v.at[pl.ds(0, 64)],
                                    acc_sh.at[pl.ds(sid * art + off, 64)])
                    return carry

                lax.fori_loop(0, art // 64, zc_body, 0)
                pltpu.sync_copy(rows0_v.at[pl.ds(0, art - 320)],
                                acc_sh.at[pl.ds(sid * art + 320,
                                                art - 320)])
                plsc.subcore_barrier()

                pltpu.async_copy(table_hbm.at[src_v.at[0]], rows0_v, sem0)

                def body(jj, carry):
                    j0 = jj * 2
                    pltpu.async_copy(table_hbm.at[src_v.at[j0 + 1]],
                                     rows1_v, sem1)
                    pltpu.make_async_copy(table_hbm.at[src_v.at[j0]],
                                          rows0_v, sem0).wait()
                    pltpu.sync_copy(rows0_v, acc_sh.at[adj_v.at[j0]],
                                    add=True)

                    @pl.when(j0 + 2 < ch)
                    def _():
                        pltpu.async_copy(table_hbm.at[src_v.at[j0 + 2]],
                                         rows0_v, sem0)

                    pltpu.make_async_copy(table_hbm.at[src_v.at[j0 + 1]],
                                          rows1_v, sem1).wait()
                    pltpu.sync_copy(rows1_v, acc_sh.at[adj_v.at[j0 + 1]],
                                    add=True)
                    return carry

                lax.fori_loop(0, ch // 2, body, 0)
                plsc.subcore_barrier()

                def wb_body(t, carry):
                    off = pl.multiple_of(t * 64, 8)
                    pltpu.sync_copy(acc_sh.at[pl.ds(sid * wrt + off, 64)],
                                    rows1_v.at[pl.ds(0, 64)])
                    pltpu.sync_copy(rows1_v.at[pl.ds(0, 64)],
                                    out_hbm.at[pl.ds(base + sid * wrt + off,
                                                     64)])
                    return carry

                lax.fori_loop(0, wrt // 64, wb_body, 0)
                plsc.subcore_barrier()

        @pl.when(cid == 0)
        def _():
            run(lo_hbm, out_lo)

        @pl.when(cid == 1)
        def _():
            run(hi_hbm, out_hi)

    return conv_kernel


def _tc1_body(nblocks, x_ref, degc_ref, dstb_ref, w1, b1, w2, b2, w3, b3,
              wc1, h_out, lo_out, hi_out, adj0_out, adj1_out,
              hsum_out, hmax_out, msum, mmax):
    i = pl.program_id(0)
    nrm = lax.rsqrt(degc_ref[...] + 1.0)
    dv = dstb_ref[...]
    trash = HALF + lax.broadcasted_iota(jnp.int32, dv.shape, 1)
    oob0 = (dv < 0) | (dv >= HALF)
    adj0_out[...] = jnp.where(oob0, trash, dv)
    d1 = dv - HALF
    oob1 = (d1 < 0) | (d1 >= HALF)
    adj1_out[...] = jnp.where(oob1, trash, d1)
    h = jnp.tanh(jnp.dot(x_ref[...], w1[...],
                         preferred_element_type=jnp.float32) + b1[...])
    h = jnp.tanh(jnp.dot(h, w2[...],
                         preferred_element_type=jnp.float32) + b2[...])
    h = jnp.tanh(jnp.dot(h, w3[...],
                         preferred_element_type=jnp.float32) + b3[...])
    hw = jnp.dot(h, wc1[...], preferred_element_type=jnp.float32)
    hwn = hw * nrm
    h_out[...] = h
    lo_out[...] = hwn[:, :HH]
    hi_out[...] = hwn[:, HH:]
    psum = jnp.sum(h, axis=0, keepdims=True)
    pmax = jnp.max(h, axis=0, keepdims=True)

    @pl.when(i == 0)
    def _():
        msum[...] = psum
        mmax[...] = pmax

    @pl.when(i > 0)
    def _():
        msum[...] = msum[...] + psum
        mmax[...] = jnp.maximum(mmax[...], pmax)

    @pl.when(i == nblocks - 1)
    def _():
        hsum_out[...] = msum[...]
        hmax_out[...] = mmax[...]


def _tc2_body(nblocks, s_lo, s_hi, g_lo, g_hi, h_ref, degc_ref, wa, wb,
              bias, lo_out, hi_out, csum_out, cmax_out, msum, mmax):
    i = pl.program_id(0)
    nrm = lax.rsqrt(degc_ref[...] + 1.0)
    s = jnp.concatenate([s_lo[...], s_hi[...]], axis=1)
    g = jnp.concatenate([g_lo[...], g_hi[...]], axis=1)
    c = jnp.tanh(nrm * (s + g) + bias[...])
    hw = (jnp.dot(c, wa[...], preferred_element_type=jnp.float32)
          + jnp.dot(h_ref[...], wb[...],
                    preferred_element_type=jnp.float32))
    hwn = hw * nrm
    lo_out[...] = hwn[:, :HH]
    hi_out[...] = hwn[:, HH:]
    psum = jnp.sum(c, axis=0, keepdims=True)
    pmax = jnp.max(c, axis=0, keepdims=True)

    @pl.when(i == 0)
    def _():
        msum[...] = psum
        mmax[...] = pmax

    @pl.when(i > 0)
    def _():
        msum[...] = msum[...] + psum
        mmax[...] = jnp.maximum(mmax[...], pmax)

    @pl.when(i == nblocks - 1)
    def _():
        csum_out[...] = msum[...]
        cmax_out[...] = mmax[...]


def _tc3_body(n_nodes, c1sum, c1max, c2sum, c2max, hsum_ref, hmax_ref,
              wp1, bp1, wp2, bp2, z_out):
    inv_n = 1.0 / n_nodes
    ro = jnp.concatenate(
        [c2sum[...] * inv_n, c1sum[...] * inv_n, hsum_ref[...] * inv_n,
         c2max[...], c1max[...], hmax_ref[...]], axis=1)
    z1 = jnp.tanh(jnp.dot(ro, wp1[...],
                          preferred_element_type=jnp.float32) + bp1[...])
    z_out[...] = jnp.dot(z1, wp2[...],
                         preferred_element_type=jnp.float32) + bp2[...]


def kernel(x, edge_index, W_pre1, b_pre1, W_pre2, b_pre2, W_pre3, b_pre3,
           W_c1, b_c1, W_c2, b_c2, W_p1, b_p1, W_p2, b_p2):
    n, d = x.shape
    e = edge_index.shape[1]
    hid = W_pre1.shape[1]
    out_dim = W_p2.shape[1]
    R = 1000
    nb = n // R

    src = edge_index[0]
    dst = edge_index[1]
    pad = EP - e
    src_p = jnp.concatenate([src, jnp.zeros((pad,), jnp.int32)])
    dst_p = jnp.concatenate([dst, jnp.full((pad,), -1, jnp.int32)])
    src_t = src_p.reshape(NS, (EP // NS) // K, K)
    dst_b = dst_p.reshape(EP // 128, 128)
    dst_d = dst.reshape(NS, (e // NS) // KD, KD)
    eb = (EP // 128) // nb

    degc = _make_deg(e)(dst_d).reshape(NP, 1)[:n]

    b1 = b_pre1.reshape(1, hid)
    b2 = b_pre2.reshape(1, hid)
    b3 = b_pre3.reshape(1, hid)
    bc1 = b_c1.reshape(1, hid)
    bc2 = b_c2.reshape(1, hid)
    bp1 = b_p1.reshape(1, hid)
    bp2 = b_p2.reshape(1, out_dim)
    wc2a = W_c2[:hid]
    wc2b = W_c2[hid:]

    full = lambda a: pl.BlockSpec(a.shape, lambda i: (0,) * a.ndim)
    rows = lambda c: pl.BlockSpec((R, c), lambda i: (i, 0))
    degs = pl.BlockSpec((R, 1), lambda i: (i, 0))
    stat = pl.BlockSpec((1, hid), lambda i: (0, 0))

    edg = pl.BlockSpec((eb, 128), lambda i: (i, 0))
    res = pl.pallas_call(
        functools.partial(_tc1_body, nb),
        grid=(nb,),
        in_specs=[rows(d), degs, edg, full(W_pre1), full(b1), full(W_pre2),
                  full(b2), full(W_pre3), full(b3), full(W_c1)],
        out_specs=[rows(hid), rows(HH), rows(HH), edg, edg, stat, stat],
        out_shape=[jax.ShapeDtypeStruct((n, hid), jnp.float32),
                   jax.ShapeDtypeStruct((n, HH), jnp.float32),
                   jax.ShapeDtypeStruct((n, HH), jnp.float32),
                   jax.ShapeDtypeStruct((EP // 128, 128), jnp.int32),
                   jax.ShapeDtypeStruct((EP // 128, 128), jnp.int32),
                   jax.ShapeDtypeStruct((1, hid), jnp.float32),
                   jax.ShapeDtypeStruct((1, hid), jnp.float32)],
        scratch_shapes=[pltpu.VMEM((1, hid), jnp.float32),
                        pltpu.VMEM((1, hid), jnp.float32)],
    )(x, degc, dst_b, W_pre1, b1, W_pre2, b2, W_pre3, b3, W_c1)
    h, g1_lo, g1_hi, adj0, adj1, hsum, hmax = res
    adj0 = adj0.reshape(NS, (EP // NS) // K, K)
    adj1 = adj1.reshape(NS, (EP // NS) // K, K)

    conv = _make_conv(EP)

    tc2 = pl.pallas_call(
        functools.partial(_tc2_body, nb),
        grid=(nb,),
        in_specs=[rows(HH), rows(HH), rows(HH), rows(HH), rows(hid), degs,
                  full(wc2a), full(wc2b), stat],
        out_specs=[rows(HH), rows(HH), stat, stat],
        out_shape=[jax.ShapeDtypeStruct((n, HH), jnp.float32),
                   jax.ShapeDtypeStruct((n, HH), jnp.float32),
                   jax.ShapeDtypeStruct((1, hid), jnp.float32),
                   jax.ShapeDtypeStruct((1, hid), jnp.float32)],
        scratch_shapes=[pltpu.VMEM((1, hid), jnp.float32),
                        pltpu.VMEM((1, hid), jnp.float32)],
    )

    s1_lo, s1_hi = conv(g1_lo, g1_hi, src_t, adj0, adj1)
    g2_lo, g2_hi, c1sum, c1max = tc2(s1_lo, s1_hi, g1_lo, g1_hi, h, degc,
                                     wc2a, wc2b, bc1)
    s2_lo, s2_hi = conv(g2_lo, g2_hi, src_t, adj0, adj1)
    _, _, c2sum, c2max = tc2(s2_lo, s2_hi, g2_lo, g2_hi, h, degc,
                             wc2a, wc2b, bc2)

    z = pl.pallas_call(
        functools.partial(_tc3_body, float(n)),
        grid=(1,),
        in_specs=[stat, stat, stat, stat, stat, stat, full(W_p1), full(bp1),
                  full(W_p2), full(bp2)],
        out_specs=[pl.BlockSpec((1, out_dim), lambda i: (0, 0))],
        out_shape=[jax.ShapeDtypeStruct((1, out_dim), jnp.float32)],
    )(c1sum, c1max, c2sum, c2max, hsum, hmax, W_p1, bp1, W_p2, bp2)[0]

    return z[0].reshape(out_dim)


# speedup vs baseline: 1.1223x; 1.0011x over previous
import functools

import jax
import jax.numpy as jnp
from jax import lax
from jax.experimental import pallas as pl
from jax.experimental.pallas import tpu as pltpu
from jax.experimental.pallas import tpu_sc as plsc

NC, NS, L = 2, 16, 16
NW = NC * NS
HH = 128
K = 128
EP = 327680
KD = 100
NP = 10240
HALF = NP // 2
AR = 5248


def _make_deg(e):
    et = e // NS
    ch = et // KD
    nrt = NP // NS
    mesh = plsc.VectorSubcoreMesh(core_axis_name="c", subcore_axis_name="s")

    @functools.partial(
        pl.kernel,
        out_type=jax.ShapeDtypeStruct((NP,), jnp.float32),
        mesh=mesh,
        scratch_types=[
            pltpu.VMEM((ch, KD), jnp.int32),
            pltpu.VMEM((128,), jnp.float32),
            pltpu.VMEM((nrt,), jnp.float32),
            pltpu.VMEM_SHARED((NP,), jnp.float32),
        ],
    )
    def deg_kernel(dst_hbm, out_hbm, dst_v, ones_v, zero_v, acc_sh):
        cid = lax.axis_index("c")
        sid = lax.axis_index("s")
        pltpu.sync_copy(dst_hbm.at[sid], dst_v)
        zeros16 = jnp.zeros((L,), jnp.float32)
        ones16 = jnp.ones((L,), jnp.float32)

        def obody(i, carry):
            ones_v[pl.ds(pl.multiple_of(i * L, L), L)] = ones16
            return carry

        lax.fori_loop(0, 128 // L, obody, 0)

        def zbody(i, carry):
            zero_v[pl.ds(pl.multiple_of(i * L, L), L)] = zeros16
            return carry

        lax.fori_loop(0, nrt // L, zbody, 0)
        pltpu.sync_copy(zero_v, acc_sh.at[pl.ds(sid * nrt, nrt)])
        plsc.subcore_barrier()

        def body(j, carry):
            pltpu.sync_copy(ones_v.at[pl.ds(0, KD)], acc_sh.at[dst_v.at[j]],
                            add=True)
            return carry

        lax.fori_loop(0, ch, body, 0)
        plsc.subcore_barrier()

        @pl.when(cid == 0)
        def _():
            pltpu.sync_copy(acc_sh.at[pl.ds(sid * nrt, nrt)], zero_v)
            pltpu.sync_copy(zero_v, out_hbm.at[pl.ds(sid * nrt, nrt)])

    return deg_kernel


def _make_conv(e):
    et = e // NS
    ch = et // K
    art = AR // NS
    wrt = HALF // NS
    mesh = plsc.VectorSubcoreMesh(core_axis_name="c", subcore_axis_name="s")

    @functools.partial(
        pl.kernel,
        out_type=(
            jax.ShapeDtypeStruct((NP, HH), jnp.float32),
            jax.ShapeDtypeStruct((NP, HH), jnp.float32),
        ),
        mesh=mesh,
        scratch_types=[
            pltpu.VMEM((ch, K), jnp.int32),
            pltpu.VMEM((ch, K), jnp.int32),
            pltpu.VMEM((K, HH), jnp.float32),
            pltpu.VMEM((K, HH), jnp.float32),
            pltpu.VMEM_SHARED((AR, HH), jnp.float32),
            pltpu.SemaphoreType.DMA,
            pltpu.SemaphoreType.DMA,
            pltpu.SemaphoreType.DMA,
            pltpu.SemaphoreType.DMA,
        ],
    )
    def conv_kernel(lo_hbm, hi_hbm, src_hbm, adj0_hbm, adj1_hbm,
                    out_lo, out_hi,
                    src_v, adj_v, rows0_v, rows1_v, acc_sh,
                    sem0, sem1, sem2, sem3):
        cid = lax.axis_index("c")
        sid = lax.axis_index("s")
        pltpu.sync_copy(src_hbm.at[sid], src_v)

        zeros16 = jnp.zeros((L,), jnp.float32)

        def run(table_hbm, out_hbm):
            for p, adj_hbm in ((0, adj0_hbm), (1, adj1_hbm)):
                base = p * HALF
                pltpu.sync_copy(adj_hbm.at[sid], adj_v)

                def zbody(i, carry):
                    for j in range(HH // L):
                        rows0_v[i, pl.ds(j * L, L)] = zeros16
                    return carry

                lax.fori_loop(0, 64, zbody, 0)

                def zc_body(t, carry):
                    off = pl.multiple_of(t * 64, 8)
                    pltpu.sync_copy(rows0_v.at[pl.ds(0, 64)],
                                    acc_sh.at[pl.ds(sid * art + off, 64)])
                    return carry

                lax.fori_loop(0, art // 64, zc_body, 0)
                pltpu.sync_copy(rows0_v.at[pl.ds(0, art - 320)],
                                acc_sh.at[pl.ds(sid * art + 320,
                                                art - 320)])
                plsc.subcore_barrier()

                pltpu.async_copy(table_hbm.at[src_v.at[0]], rows0_v, sem0)

                def body(jj, carry):
                    j0 = jj * 2
                    pltpu.async_copy(table_hbm.at[src_v.at[j0 + 1]],
                                     rows1_v, sem1)
                    pltpu.make_async_copy(table_hbm.at[src_v.at[j0]],
                                          rows0_v, sem0).wait()
                    pltpu.sync_copy(rows0_v, acc_sh.at[adj_v.at[j0]],
                                    add=True)

                    @pl.when(j0 + 2 < ch)
                    def _():
                        pltpu.async_copy(table_hbm.at[src_v.at[j0 + 2]],
                                         rows0_v, sem0)

                    pltpu.make_async_copy(table_hbm.at[src_v.at[j0 + 1]],
                                          rows1_v, sem1).wait()
                    pltpu.sync_copy(rows1_v, acc_sh.at[adj_v.at[j0 + 1]],
                                    add=True)
                    return carry

                lax.fori_loop(0, ch // 2, body, 0)
                plsc.subcore_barrier()

                pltpu.sync_copy(acc_sh.at[pl.ds(sid * wrt, wrt)],
                                out_hbm.at[pl.ds(base + sid * wrt, wrt)])
                plsc.subcore_barrier()

        @pl.when(cid == 0)
        def _():
            run(lo_hbm, out_lo)

        @pl.when(cid == 1)
        def _():
            run(hi_hbm, out_hi)

    return conv_kernel


def _tc1_body(nblocks, x_ref, degc_ref, dstb_ref, w1, b1, w2, b2, w3, b3,
              wc1, h_out, lo_out, hi_out, adj0_out, adj1_out,
              hsum_out, hmax_out, msum, mmax):
    i = pl.program_id(0)
    nrm = lax.rsqrt(degc_ref[...] + 1.0)
    dv = dstb_ref[...]
    trash = HALF + lax.broadcasted_iota(jnp.int32, dv.shape, 1)
    oob0 = (dv < 0) | (dv >= HALF)
    adj0_out[...] = jnp.where(oob0, trash, dv)
    d1 = dv - HALF
    oob1 = (d1 < 0) | (d1 >= HALF)
    adj1_out[...] = jnp.where(oob1, trash, d1)
    h = jnp.tanh(jnp.dot(x_ref[...], w1[...],
                         preferred_element_type=jnp.float32) + b1[...])
    h = jnp.tanh(jnp.dot(h, w2[...],
                         preferred_element_type=jnp.float32) + b2[...])
    h = jnp.tanh(jnp.dot(h, w3[...],
                         preferred_element_type=jnp.float32) + b3[...])
    hw = jnp.dot(h, wc1[...], preferred_element_type=jnp.float32)
    hwn = hw * nrm
    h_out[...] = h
    lo_out[...] = hwn[:, :HH]
    hi_out[...] = hwn[:, HH:]
    psum = jnp.sum(h, axis=0, keepdims=True)
    pmax = jnp.max(h, axis=0, keepdims=True)

    @pl.when(i == 0)
    def _():
        msum[...] = psum
        mmax[...] = pmax

    @pl.when(i > 0)
    def _():
        msum[...] = msum[...] + psum
        mmax[...] = jnp.maximum(mmax[...], pmax)

    @pl.when(i == nblocks - 1)
    def _():
        hsum_out[...] = msum[...]
        hmax_out[...] = mmax[...]


def _tc2_body(nblocks, s_lo, s_hi, g_lo, g_hi, h_ref, degc_ref, wa, wb,
              bias, lo_out, hi_out, csum_out, cmax_out, msum, mmax):
    i = pl.program_id(0)
    nrm = lax.rsqrt(degc_ref[...] + 1.0)
    s = jnp.concatenate([s_lo[...], s_hi[...]], axis=1)
    g = jnp.concatenate([g_lo[...], g_hi[...]], axis=1)
    c = jnp.tanh(nrm * (s + g) + bias[...])
    hw = (jnp.dot(c, wa[...], preferred_element_type=jnp.float32)
          + jnp.dot(h_ref[...], wb[...],
                    preferred_element_type=jnp.float32))
    hwn = hw * nrm
    lo_out[...] = hwn[:, :HH]
    hi_out[...] = hwn[:, HH:]
    psum = jnp.sum(c, axis=0, keepdims=True)
    pmax = jnp.max(c, axis=0, keepdims=True)

    @pl.when(i == 0)
    def _():
        msum[...] = psum
        mmax[...] = pmax

    @pl.when(i > 0)
    def _():
        msum[...] = msum[...] + psum
        mmax[...] = jnp.maximum(mmax[...], pmax)

    @pl.when(i == nblocks - 1)
    def _():
        csum_out[...] = msum[...]
        cmax_out[...] = mmax[...]


def _tc3_body(n_nodes, c1sum, c1max, c2sum, c2max, hsum_ref, hmax_ref,
              wp1, bp1, wp2, bp2, z_out):
    inv_n = 1.0 / n_nodes
    ro = jnp.concatenate(
        [c2sum[...] * inv_n, c1sum[...] * inv_n, hsum_ref[...] * inv_n,
         c2max[...], c1max[...], hmax_ref[...]], axis=1)
    z1 = jnp.tanh(jnp.dot(ro, wp1[...],
                          preferred_element_type=jnp.float32) + bp1[...])
    z_out[...] = jnp.dot(z1, wp2[...],
                         preferred_element_type=jnp.float32) + bp2[...]


def kernel(x, edge_index, W_pre1, b_pre1, W_pre2, b_pre2, W_pre3, b_pre3,
           W_c1, b_c1, W_c2, b_c2, W_p1, b_p1, W_p2, b_p2):
    n, d = x.shape
    e = edge_index.shape[1]
    hid = W_pre1.shape[1]
    out_dim = W_p2.shape[1]
    R = 1000
    nb = n // R

    src = edge_index[0]
    dst = edge_index[1]
    pad = EP - e
    src_p = jnp.concatenate([src, jnp.zeros((pad,), jnp.int32)])
    dst_p = jnp.concatenate([dst, jnp.full((pad,), -1, jnp.int32)])
    src_t = src_p.reshape(NS, (EP // NS) // K, K)
    dst_b = dst_p.reshape(EP // 128, 128)
    dst_d = dst.reshape(NS, (e // NS) // KD, KD)
    eb = (EP // 128) // nb

    degc = _make_deg(e)(dst_d).reshape(NP, 1)[:n]

    b1 = b_pre1.reshape(1, hid)
    b2 = b_pre2.reshape(1, hid)
    b3 = b_pre3.reshape(1, hid)
    bc1 = b_c1.reshape(1, hid)
    bc2 = b_c2.reshape(1, hid)
    bp1 = b_p1.reshape(1, hid)
    bp2 = b_p2.reshape(1, out_dim)
    wc2a = W_c2[:hid]
    wc2b = W_c2[hid:]

    full = lambda a: pl.BlockSpec(a.shape, lambda i: (0,) * a.ndim)
    rows = lambda c: pl.BlockSpec((R, c), lambda i: (i, 0))
    degs = pl.BlockSpec((R, 1), lambda i: (i, 0))
    stat = pl.BlockSpec((1, hid), lambda i: (0, 0))

    edg = pl.BlockSpec((eb, 128), lambda i: (i, 0))
    res = pl.pallas_call(
        functools.partial(_tc1_body, nb),
        grid=(nb,),
        in_specs=[rows(d), degs, edg, full(W_pre1), full(b1), full(W_pre2),
                  full(b2), full(W_pre3), full(b3), full(W_c1)],
        out_specs=[rows(hid), rows(HH), rows(HH), edg, edg, stat, stat],
        out_shape=[jax.ShapeDtypeStruct((n, hid), jnp.float32),
                   jax.ShapeDtypeStruct((n, HH), jnp.float32),
                   jax.ShapeDtypeStruct((n, HH), jnp.float32),
                   jax.ShapeDtypeStruct((EP // 128, 128), jnp.int32),
                   jax.ShapeDtypeStruct((EP // 128, 128), jnp.int32),
                   jax.ShapeDtypeStruct((1, hid), jnp.float32),
                   jax.ShapeDtypeStruct((1, hid), jnp.float32)],
        scratch_shapes=[pltpu.VMEM((1, hid), jnp.float32),
                        pltpu.VMEM((1, hid), jnp.float32)],
    )(x, degc, dst_b, W_pre1, b1, W_pre2, b2, W_pre3, b3, W_c1)
    h, g1_lo, g1_hi, adj0, adj1, hsum, hmax = res
    adj0 = adj0.reshape(NS, (EP // NS) // K, K)
    adj1 = adj1.reshape(NS, (EP // NS) // K, K)

    conv = _make_conv(EP)

    tc2 = pl.pallas_call(
        functools.partial(_tc2_body, nb),
        grid=(nb,),
        in_specs=[rows(HH), rows(HH), rows(HH), rows(HH), rows(hid), degs,
                  full(wc2a), full(wc2b), stat],
        out_specs=[rows(HH), rows(HH), stat, stat],
        out_shape=[jax.ShapeDtypeStruct((n, HH), jnp.float32),
                   jax.ShapeDtypeStruct((n, HH), jnp.float32),
                   jax.ShapeDtypeStruct((1, hid), jnp.float32),
                   jax.ShapeDtypeStruct((1, hid), jnp.float32)],
        scratch_shapes=[pltpu.VMEM((1, hid), jnp.float32),
                        pltpu.VMEM((1, hid), jnp.float32)],
    )

    s1_lo, s1_hi = conv(g1_lo, g1_hi, src_t, adj0, adj1)
    g2_lo, g2_hi, c1sum, c1max = tc2(s1_lo, s1_hi, g1_lo, g1_hi, h, degc,
                                     wc2a, wc2b, bc1)
    s2_lo, s2_hi = conv(g2_lo, g2_hi, src_t, adj0, adj1)
    _, _, c2sum, c2max = tc2(s2_lo, s2_hi, g2_lo, g2_hi, h, degc,
                             wc2a, wc2b, bc2)

    z = pl.pallas_call(
        functools.partial(_tc3_body, float(n)),
        grid=(1,),
        in_specs=[stat, stat, stat, stat, stat, stat, full(W_p1), full(bp1),
                  full(W_p2), full(bp2)],
        out_specs=[pl.BlockSpec((1, out_dim), lambda i: (0, 0))],
        out_shape=[jax.ShapeDtypeStruct((1, out_dim), jnp.float32)],
    )(c1sum, c1max, c2sum, c2max, hsum, hmax, W_p1, bp1, W_p2, bp2)[0]

    return z[0].reshape(out_dim)
